# Initial kernel scaffold; baseline (speedup 1.0000x reference)
import jax
import jax.numpy as jnp
from jax.experimental import pallas as pl


def kernel(x, edge_index, batch, W1, a1s, a1d, b1, W2, a2s, a2d, b2, W4, a4s, a4d, b4, Wl, bl):
    raise NotImplementedError("write your pallas kernel here")


# speedup vs baseline: 6.3163x; 6.3163x over previous
import functools

import jax
import jax.numpy as jnp
from jax import lax
from jax.experimental import pallas as pl
from jax.experimental.pallas import tpu as pltpu
from jax.experimental.pallas import tpu_sc as plsc

N = 50000
E = 800000
NG = 128
R = 1000
GRID = N // R
SUB = 80


def _lrelu(z):
    return jnp.where(z > 0, z, 0.2 * z)


def _make_sc_pass(Hs, W, Wn, C, shift, n_heads_local, col_head, mask_cols,
                  want, qpass=0, nq=2):
    K = C // SUB
    NCHUNK = (E // 16) // C
    VG = C // 16
    ZR = 3128
    ZR_LAST = N - 15 * ZR

    WA = Wn if want == "num" else 4

    mesh = plsc.VectorSubcoreMesh(core_axis_name="c", subcore_axis_name="s")

    scratch = [
        pltpu.VMEM((K, SUB), jnp.int32),
        pltpu.VMEM((K, SUB), jnp.int32),
        pltpu.VMEM((C, Hs), jnp.float32),
        pltpu.VMEM((C, Hs), jnp.float32),
        pltpu.VMEM((C, 4), jnp.float32),
    ]
    if want == "num":
        scratch += [
            pltpu.VMEM((K, SUB), jnp.int32),
            pltpu.VMEM((C, W), jnp.float32),
        ]
    scratch.append(pltpu.VMEM_SHARED((N, WA), jnp.float32))

    @functools.partial(
        pl.kernel,
        mesh=mesh,
        compiler_params=pltpu.CompilerParams(needs_layout_passes=False,
                                             use_tc_tiling_on_sc=False),
        out_type=jax.ShapeDtypeStruct((2 * N, WA), jnp.float32),
        scratch_types=scratch,
    )
    def sc_kernel(*refs):
        if want == "num":
            (src_h, dst_h, as_h, ad_h, hm_h, zacc_h, acc_out,
             sidx, didx, asrows, adrows, exrows, gidx, hrows, acc_sh) = refs
        else:
            (src_h, dst_h, as_h, ad_h, zacc_h, acc_out,
             sidx, didx, asrows, adrows, exrows, acc_sh) = refs
        cid = lax.axis_index("c")
        sid = lax.axis_index("s")

        @pl.when(sid < 15)
        def _():
            pltpu.sync_copy(zacc_h.at[pl.ds(0, ZR)],
                            acc_sh.at[pl.ds(sid * ZR, ZR)])

        @pl.when(sid == 15)
        def _():
            pltpu.sync_copy(zacc_h.at[pl.ds(0, ZR_LAST)],
                            acc_sh.at[pl.ds(15 * ZR, ZR_LAST)])

        plsc.subcore_barrier()

        zero16 = jnp.zeros((16,), jnp.float32)
        base_iota = lax.broadcasted_iota(jnp.int32, (16,), 0)

        def chunk_body(it, _):
            ebase = sid * (E // 16) + it * C
            for j in range(K):
                pltpu.sync_copy(src_h.at[pl.ds(ebase + j * SUB, SUB)],
                                sidx.at[j])
                pltpu.sync_copy(dst_h.at[pl.ds(ebase + j * SUB, SUB)],
                                didx.at[j])
            for j in range(K):
                pltpu.sync_copy(as_h.at[sidx.at[j]],
                                asrows.at[pl.ds(j * SUB, SUB)])
            for j in range(K):
                pltpu.sync_copy(ad_h.at[didx.at[j]],
                                adrows.at[pl.ds(j * SUB, SUB)])
            if want == "num":
                if shift:
                    qid = qpass * 2 + cid
                    def mk_gidx(j, _):
                        for v in range(SUB // 16):
                            s = sidx[j, pl.ds(v * 16, 16)]
                            gidx[j, pl.ds(v * 16, 16)] = s + qid * N
                        return _
                    lax.fori_loop(0, K, mk_gidx, 0)
                    hsrc = gidx
                else:
                    hsrc = sidx
                for j in range(K):
                    pltpu.sync_copy(hm_h.at[hsrc.at[j]],
                                    hrows.at[pl.ds(j * SUB, SUB)])

            def ex_body(v, _):
                rows16 = base_iota + v * 16
                for lh in range(4):
                    if lh < n_heads_local:
                        gh = (lh + n_heads_local * (2 * qpass + cid)
                              if Hs > 1 else 0)
                        a = plsc.load_gather(
                            asrows, [rows16, jnp.full((16,), gh, jnp.int32)])
                        b = plsc.load_gather(
                            adrows, [rows16, jnp.full((16,), gh, jnp.int32)])
                        e = jnp.exp(_lrelu(a + b))
                    else:
                        e = zero16
                    plsc.store_scatter(
                        exrows, [rows16, jnp.full((16,), lh, jnp.int32)], e)
                return _
            lax.fori_loop(0, VG, ex_body, 0)

            if want == "num":
                def mul_body(v, _):
                    rows16 = base_iota + v * 16
                    for c in range(W):
                        exc = plsc.load_gather(
                            exrows,
                            [rows16, jnp.full((16,), col_head(c), jnp.int32)])
                        if mask_cols:
                            w = jnp.where(cid == (c // 2), 1.0, 0.0)
                            exc = exc * w
                        hcol = plsc.load_gather(
                            hrows, [rows16, jnp.full((16,), c, jnp.int32)])
                        plsc.store_scatter(
                            hrows, [rows16, jnp.full((16,), c, jnp.int32)],
                            exc * hcol)
                    return _
                lax.fori_loop(0, VG, mul_body, 0)

                for j in range(K):
                    pltpu.sync_copy(hrows.at[pl.ds(j * SUB, SUB)],
                                    acc_sh.at[didx.at[j]], add=True)
            else:
                for j in range(K):
                    pltpu.sync_copy(exrows.at[pl.ds(j * SUB, SUB)],
                                    acc_sh.at[didx.at[j]], add=True)
            return _

        lax.fori_loop(0, NCHUNK, chunk_body, 0)
        plsc.subcore_barrier()

        @pl.when(sid < 15)
        def _():
            pltpu.sync_copy(acc_sh.at[pl.ds(sid * ZR, ZR)],
                            acc_out.at[pl.ds(cid * N + sid * ZR, ZR)])

        @pl.when(sid == 15)
        def _():
            pltpu.sync_copy(acc_sh.at[pl.ds(15 * ZR, ZR_LAST)],
                            acc_out.at[pl.ds(cid * N + 15 * ZR, ZR_LAST)])

    return sc_kernel


def _tc_pre1(x, W1, A1s, A1d):
    def body(x_r, w_r, as_r, ad_r, h_r, asr_r, adr_r):
        h = jnp.dot(x_r[...], w_r[...], preferred_element_type=jnp.float32)
        h_r[...] = h
        asr_r[...] = jnp.dot(h, as_r[...], preferred_element_type=jnp.float32)
        adr_r[...] = jnp.dot(h, ad_r[...], preferred_element_type=jnp.float32)

    return pl.pallas_call(
        body,
        grid=(GRID,),
        in_specs=[
            pl.BlockSpec((R, 4), lambda i: (i, 0)),
            pl.BlockSpec((4, 64), lambda i: (0, 0)),
            pl.BlockSpec((64, 8), lambda i: (0, 0)),
            pl.BlockSpec((64, 8), lambda i: (0, 0)),
        ],
        out_specs=[
            pl.BlockSpec((R, 64), lambda i: (i, 0)),
            pl.BlockSpec((R, 8), lambda i: (i, 0)),
            pl.BlockSpec((R, 8), lambda i: (i, 0)),
        ],
        out_shape=[
            jax.ShapeDtypeStruct((N, 64), jnp.float32),
            jax.ShapeDtypeStruct((N, 8), jnp.float32),
            jax.ShapeDtypeStruct((N, 8), jnp.float32),
        ],
    )(x, W1, A1s, A1d)


_NUM_SPECS = [
    pl.BlockSpec((R, 8), (lambda q: (lambda i: (i + q * GRID, 0)))(q % 2))
    for q in range(8)
]


def _tc_comb1(asr, adr, h, nums, den01, b1, W2, a2sv, a2dv):
    def body(as_r, ad_r, h_r, n0_r, n1_r, n2_r, n3_r, n4_r, n5_r, n6_r, n7_r,
             d0_r, d1_r, b_r, w_r, avs_r, avd_r, o_r, as2_r, ad2_r):
        ex = jnp.exp(_lrelu(as_r[...] + ad_r[...]))
        h = h_r[...]
        ns = [n0_r, n1_r, n2_r, n3_r, n4_r, n5_r, n6_r, n7_r]
        cols = []
        for g in range(8):
            d_half = d0_r[...] if g < 4 else d1_r[...]
            hg = h[:, 8 * g:8 * g + 8]
            eg = ex[:, g:g + 1]
            num_g = ns[g][...] + eg * hg
            den_g = d_half[:, (g % 4):(g % 4) + 1] + eg
            cols.append(num_g / den_g)
        out = jnp.concatenate(cols, axis=1) + b_r[...]
        act = jnp.where(out > 0, out, jnp.exp(out) - 1.0)
        h2 = jnp.dot(act, w_r[...], preferred_element_type=jnp.float32)
        o_r[...] = h2
        as2_r[...] = jnp.dot(h2, avs_r[...], preferred_element_type=jnp.float32)
        ad2_r[...] = jnp.dot(h2, avd_r[...], preferred_element_type=jnp.float32)

    return pl.pallas_call(
        body,
        grid=(GRID,),
        in_specs=[
            pl.BlockSpec((R, 8), lambda i: (i, 0)),
            pl.BlockSpec((R, 8), lambda i: (i, 0)),
            pl.BlockSpec((R, 64), lambda i: (i, 0)),
        ] + list(_NUM_SPECS) + [
            pl.BlockSpec((R, 4), lambda i: (i, 0)),
            pl.BlockSpec((R, 4), lambda i: (i + GRID, 0)),
            pl.BlockSpec((1, 64), lambda i: (0, 0)),
            pl.BlockSpec((64, 64), lambda i: (0, 0)),
            pl.BlockSpec((64, 1), lambda i: (0, 0)),
            pl.BlockSpec((64, 1), lambda i: (0, 0)),
        ],
        out_specs=[
            pl.BlockSpec((R, 64), lambda i: (i, 0)),
            pl.BlockSpec((R, 1), lambda i: (i, 0)),
            pl.BlockSpec((R, 1), lambda i: (i, 0)),
        ],
        out_shape=[
            jax.ShapeDtypeStruct((N, 64), jnp.float32),
            jax.ShapeDtypeStruct((N, 1), jnp.float32),
            jax.ShapeDtypeStruct((N, 1), jnp.float32),
        ],
    )(asr, adr, h, nums[0], nums[0], nums[1], nums[1],
      nums[2], nums[2], nums[3], nums[3], den01, den01,
      b1, W2, a2sv, a2dv)


def _tc_comb_mid(asv, adv, h, nums, den01, b2, Wn, ansv, andv, wn_out):
    def body(as_r, ad_r, h_r, n0_r, n1_r, n2_r, n3_r, n4_r, n5_r, n6_r, n7_r,
             d_r, b_r, w_r, avs_r, avd_r, ha_r, asn_r, adn_r):
        ex = jnp.exp(_lrelu(as_r[...] + ad_r[...]))
        ns = [n0_r, n1_r, n2_r, n3_r, n4_r, n5_r, n6_r, n7_r]
        num = jnp.concatenate([r[...] for r in ns], axis=1) + ex * h_r[...]
        den = d_r[:, 0:1] + ex
        out = num / den + b_r[...]
        hn = jnp.dot(out, w_r[...], preferred_element_type=jnp.float32)
        ha_r[...] = hn
        asn_r[...] = jnp.dot(hn, avs_r[...], preferred_element_type=jnp.float32)
        adn_r[...] = jnp.dot(hn, avd_r[...], preferred_element_type=jnp.float32)

    return pl.pallas_call(
        body,
        grid=(GRID,),
        in_specs=[
            pl.BlockSpec((R, 1), lambda i: (i, 0)),
            pl.BlockSpec((R, 1), lambda i: (i, 0)),
            pl.BlockSpec((R, 64), lambda i: (i, 0)),
        ] + list(_NUM_SPECS) + [
            pl.BlockSpec((R, 4), lambda i: (i, 0)),
            pl.BlockSpec((1, 64), lambda i: (0, 0)),
            pl.BlockSpec((64, wn_out), lambda i: (0, 0)),
            pl.BlockSpec((wn_out, 1), lambda i: (0, 0)),
            pl.BlockSpec((wn_out, 1), lambda i: (0, 0)),
        ],
        out_specs=[
            pl.BlockSpec((R, wn_out), lambda i: (i, 0)),
            pl.BlockSpec((R, 1), lambda i: (i, 0)),
            pl.BlockSpec((R, 1), lambda i: (i, 0)),
        ],
        out_shape=[
            jax.ShapeDtypeStruct((N, wn_out), jnp.float32),
            jax.ShapeDtypeStruct((N, 1), jnp.float32),
            jax.ShapeDtypeStruct((N, 1), jnp.float32),
        ],
    )(asv, adv, h, nums[0], nums[0], nums[1], nums[1],
      nums[2], nums[2], nums[3], nums[3], den01,
      b2, Wn, ansv, andv)


def _tc_final(num01, den01, as4, ad4, h4pre, b4, batch2, Wl, bl):
    def body(n0_r, n1_r, d_r, as_r, ad_r, hp_r, b_r, bt_r, wl_r, bl_r,
             out_r, sums, cnt):
        i = pl.program_id(0)

        @pl.when(i == 0)
        def _():
            sums[...] = jnp.zeros_like(sums)
            cnt[...] = jnp.zeros_like(cnt)

        ex = jnp.exp(_lrelu(as_r[...] + ad_r[...]))
        num = n0_r[...] + n1_r[...] + ex * hp_r[...]
        den = d_r[:, 0:1] + ex
        h4 = num / den + b_r[...]

        bt = bt_r[...][:, 0]
        oh = (lax.broadcasted_iota(jnp.int32, (NG, R), 0)
              == bt[None, :]).astype(jnp.float32)
        sums[...] += jnp.dot(oh, h4, preferred_element_type=jnp.float32)
        cnt[...] += jnp.sum(oh, axis=1, keepdims=True)

        @pl.when(i == GRID - 1)
        def _():
            pooled = sums[...] / jnp.maximum(cnt[...], 1.0)
            out_r[...] = (jnp.dot(pooled, wl_r[...],
                                  preferred_element_type=jnp.float32)
                          + bl_r[...])

    return pl.pallas_call(
        body,
        grid=(GRID,),
        in_specs=[
            pl.BlockSpec((R, 4), lambda i: (i, 0)),
            pl.BlockSpec((R, 4), lambda i: (i + GRID, 0)),
            pl.BlockSpec((R, 4), lambda i: (i, 0)),
            pl.BlockSpec((R, 1), lambda i: (i, 0)),
            pl.BlockSpec((R, 1), lambda i: (i, 0)),
            pl.BlockSpec((R, 4), lambda i: (i, 0)),
            pl.BlockSpec((1, 4), lambda i: (0, 0)),
            pl.BlockSpec((R, 1), lambda i: (i, 0)),
            pl.BlockSpec((4, 1), lambda i: (0, 0)),
            pl.BlockSpec((1, 1), lambda i: (0, 0)),
        ],
        out_specs=pl.BlockSpec((NG, 1), lambda i: (0, 0)),
        out_shape=jax.ShapeDtypeStruct((NG, 1), jnp.float32),
        scratch_shapes=[
            pltpu.VMEM((NG, 4), jnp.float32),
            pltpu.VMEM((NG, 1), jnp.float32),
        ],
    )(num01, num01, den01, as4, ad4, h4pre, b4, batch2, Wl, bl)


_sc1_den = _make_sc_pass(Hs=8, W=8, Wn=4, C=400, shift=True,
                         n_heads_local=4, col_head=lambda c: 0,
                         mask_cols=False, want="den")
_sc1_num = [
    _make_sc_pass(Hs=8, W=8, Wn=8, C=400, shift=True,
                  n_heads_local=1, col_head=lambda c: 0,
                  mask_cols=False, want="num", qpass=q)
    for q in range(4)
]
_sc23_den = _make_sc_pass(Hs=1, W=8, Wn=4, C=2000, shift=True,
                          n_heads_local=1, col_head=lambda c: 0,
                          mask_cols=False, want="den")
_sc23_num = [
    _make_sc_pass(Hs=1, W=8, Wn=8, C=2000, shift=True,
                  n_heads_local=1, col_head=lambda c: 0,
                  mask_cols=False, want="num", qpass=q)
    for q in range(4)
]
_sc4_den = _make_sc_pass(Hs=1, W=4, Wn=4, C=2000, shift=False,
                         n_heads_local=1, col_head=lambda c: 0,
                         mask_cols=True, want="den")
_sc4_num = _make_sc_pass(Hs=1, W=4, Wn=4, C=2000, shift=False,
                         n_heads_local=1, col_head=lambda c: 0,
                         mask_cols=True, want="num")


def kernel(x, edge_index, batch, W1, a1s, a1d, b1, W2, a2s, a2d, b2,
           W4, a4s, a4d, b4, Wl, bl):
    src = edge_index[0]
    dst = edge_index[1]

    eye8 = jnp.eye(8, dtype=jnp.float32)
    A1s = (a1s[0][:, :, None] * eye8[:, None, :]).reshape(64, 8)
    A1d = (a1d[0][:, :, None] * eye8[:, None, :]).reshape(64, 8)
    a2sv = a2s.reshape(64, 1)
    a2dv = a2d.reshape(64, 1)
    a4sv = a4s.reshape(4, 1)
    a4dv = a4d.reshape(4, 1)
    b1r = b1.reshape(1, 64)
    b2r = b2.reshape(1, 64)
    b4r = b4.reshape(1, 4)
    blr = bl.reshape(1, 1)
    batch2 = batch.reshape(N, 1)
    znum8 = jnp.zeros((3128, 8), jnp.float32)
    znum4 = jnp.zeros((3128, 4), jnp.float32)
    zden = jnp.zeros((3128, 4), jnp.float32)

    def split8(h):
        return jnp.concatenate([h[:, 8 * q:8 * q + 8] for q in range(8)],
                               axis=0)

    h, asr, adr = _tc_pre1(x, W1, A1s, A1d)
    h8 = split8(h)
    den01 = _sc1_den(src, dst, asr, adr, zden)
    nums = [f(src, dst, asr, adr, h8, znum8) for f in _sc1_num]
    h, as2, ad2 = _tc_comb1(asr, adr, h, nums, den01, b1r, W2, a2sv, a2dv)

    h8 = split8(h)
    den01 = _sc23_den(src, dst, as2, ad2, zden)
    nums = [f(src, dst, as2, ad2, h8, znum8) for f in _sc23_num]
    h, as3, ad3 = _tc_comb_mid(as2, ad2, h, nums, den01,
                               b2r, W2, a2sv, a2dv, 64)

    h8 = split8(h)
    den01 = _sc23_den(src, dst, as3, ad3, zden)
    nums = [f(src, dst, as3, ad3, h8, znum8) for f in _sc23_num]
    h4pre, as4, ad4 = _tc_comb_mid(as3, ad3, h, nums, den01,
                                   b2r, W4, a4sv, a4dv, 4)

    den01 = _sc4_den(src, dst, as4, ad4, zden)
    num01 = _sc4_num(src, dst, as4, ad4, h4pre, znum4)
    return _tc_final(num01, den01, as4, ad4, h4pre, b4r, batch2, Wl, blr)
